# baseline (speedup 1.0000x reference)
import functools

import jax
import jax.numpy as jnp
from jax import lax
from jax.experimental import pallas as pl
from jax.experimental.pallas import tpu as pltpu
from jax.experimental.pallas import tpu_sc as plsc

_B, _S, _D = 16, 2048, 512
_NV = 32
_T = _S + _NV
_HALF = _T // 2
_ZR = 128
_AUXR = _NV + _ZR
_LEVELS = (128, 16, 2, 1)


def _virt_body(tok_ref, w1_ref, b1_ref, w2_ref, b2_ref, aux_ref):
    h = jnp.tanh(
        jnp.dot(tok_ref[...], w1_ref[...], preferred_element_type=jnp.float32)
        + b1_ref[...]
    )
    aux_ref[: _NV, :] = (
        jnp.dot(h, w2_ref[...], preferred_element_type=jnp.float32) + b2_ref[...]
    )
    aux_ref[_NV :, :] = jnp.zeros((_ZR, _D), jnp.float32)


_tc_virtual = pl.pallas_call(
    _virt_body,
    out_shape=jax.ShapeDtypeStruct((_AUXR, _D), jnp.float32),
)


@functools.partial(
    pl.kernel,
    out_type=jax.ShapeDtypeStruct((_B * _T * _D,), jnp.float32),
    mesh=plsc.VectorSubcoreMesh(core_axis_name="c", subcore_axis_name="s"),
    scratch_types=[
        pltpu.VMEM((2 * _B,), jnp.int32),
        pltpu.VMEM((_AUXR * _D,), jnp.float32),
    ],
)
def _sc_assemble(in_hbm, aux_hbm, seq_hbm, out_hbm, len_v, aux_v):
    c = lax.axis_index("c")
    s = lax.axis_index("s")
    b = s
    t0 = c * _HALF

    pltpu.sync_copy(seq_hbm, len_v.at[pl.ds(0, _B)])
    pltpu.sync_copy(aux_hbm, aux_v)

    L = len_v[pl.ds(b, _B)][0]

    v_lo = jnp.clip(L - t0, 0, _HALF)
    v_hi = jnp.clip(L + _NV - t0, 0, _HALF)
    vsrc0 = t0 + v_lo - L
    dst0 = (b * _T + t0) * _D

    def run_levels(n_rows, make_src, dst_base):
        done = jnp.int32(0)
        rem = n_rows
        for lv in _LEVELS:
            cnt = lax.div(rem, jnp.int32(lv))

            def body(i, carry, lv=lv, done=done):
                r = done + i * lv
                pltpu.sync_copy(
                    make_src(r, lv),
                    out_hbm.at[pl.ds(dst0 + (dst_base + r) * _D, lv * _D)],
                )
                return carry

            lax.fori_loop(0, cnt, body, jnp.int32(0))
            done = done + cnt * lv
            rem = rem - cnt * lv

    src0 = (b * _S + t0) * _D
    run_levels(
        v_lo, lambda r, lv: in_hbm.at[pl.ds(src0 + r * _D, lv * _D)], jnp.int32(0)
    )
    run_levels(
        v_hi - v_lo,
        lambda r, lv: aux_v.at[pl.ds((vsrc0 + r) * _D, lv * _D)],
        v_lo,
    )
    run_levels(
        _HALF - v_hi, lambda r, lv: aux_v.at[pl.ds(_NV * _D, lv * _D)], v_hi
    )


def kernel(inputs, seq_len, embed_table, W1, b1, W2, b2):
    aux = _tc_virtual(
        embed_table, W1, b1.reshape(1, -1), W2, b2.reshape(1, -1)
    )
    out = _sc_assemble(inputs.reshape(-1), aux.reshape(-1), seq_len)
    return out.reshape(_B, _T, _D), seq_len + _NV
